# Initial kernel scaffold; baseline (speedup 1.0000x reference)
import jax
import jax.numpy as jnp
from jax.experimental import pallas as pl


def kernel(x, edge_index, batch, W1a, b1a, W1b, b1b, W2a, b2a, W2b, b2b, Wfc, bfc):
    raise NotImplementedError("write your pallas kernel here")


# speedup vs baseline: 11.7513x; 11.7513x over previous
import functools

import jax
import jax.numpy as jnp
from jax import lax
from jax.experimental import pallas as pl
from jax.experimental.pallas import tpu as pltpu, tpu_sc as plsc

F_PAD = 16


def _sc_segment_sum(p, src2d, dst2d):
  n = p.shape[0]
  nchunk, k = src2d.shape
  nc, ns = 2, 16
  cpt = nchunk // (nc * ns)
  rows_a = 632
  rows_b = n - 15 * rows_a

  mesh = plsc.VectorSubcoreMesh(core_axis_name="c", subcore_axis_name="s")

  @functools.partial(
      pl.kernel,
      out_type=jax.ShapeDtypeStruct((nc, n, F_PAD), jnp.float32),
      mesh=mesh,
      compiler_params=pltpu.CompilerParams(use_tc_tiling_on_sc=False),
      scratch_types=[
          pltpu.VMEM((cpt, k), jnp.int32),
          pltpu.VMEM((cpt, k), jnp.int32),
          pltpu.VMEM((k, F_PAD), jnp.float32),
          pltpu.VMEM((rows_a, F_PAD), jnp.float32),
          pltpu.VMEM_SHARED((n, F_PAD), jnp.float32),
          pltpu.SemaphoreType.DMA,
      ],
  )
  def k_fn(p_hbm, src_hbm, dst_hbm, out_hbm, sidx, didx, rows, zbuf, acc, sem):
    c = lax.axis_index("c")
    s = lax.axis_index("s")
    tile = c * ns + s

    def zrow(i, carry):
      zbuf[i, :] = jnp.zeros((F_PAD,), jnp.float32)
      return carry

    lax.fori_loop(0, rows_a, zrow, 0)

    @pl.when(s < ns - 1)
    def _():
      pltpu.sync_copy(zbuf.at[pl.ds(0, rows_a)],
                      acc.at[pl.ds(s * rows_a, rows_a)])

    @pl.when(s == ns - 1)
    def _():
      pltpu.sync_copy(zbuf.at[pl.ds(0, rows_b)],
                      acc.at[pl.ds((ns - 1) * rows_a, rows_b)])

    pltpu.sync_copy(src_hbm.at[pl.ds(tile * cpt, cpt)], sidx)
    pltpu.sync_copy(dst_hbm.at[pl.ds(tile * cpt, cpt)], didx)
    plsc.subcore_barrier()

    def chunk(j, carry):
      pltpu.async_copy(p_hbm.at[sidx.at[j]], rows, sem).wait()
      pltpu.sync_copy(rows, acc.at[didx.at[j]], add=True)
      return carry

    lax.fori_loop(0, cpt, chunk, 0)
    plsc.subcore_barrier()

    @pl.when(s < ns - 1)
    def _():
      pltpu.sync_copy(acc.at[pl.ds(s * rows_a, rows_a)],
                      out_hbm.at[c, pl.ds(s * rows_a, rows_a)])

    @pl.when(s == ns - 1)
    def _():
      pltpu.sync_copy(acc.at[pl.ds((ns - 1) * rows_a, rows_b)],
                      out_hbm.at[c, pl.ds((ns - 1) * rows_a, rows_b)])

  return k_fn(p, src2d, dst2d)


_BLK = 1000


def _proj_body(x_ref, w_ref, o_ref):
  o_ref[...] = jnp.dot(x_ref[...], w_ref[...],
                       preferred_element_type=jnp.float32)


def _proj(x, w1a):
  n = x.shape[0]
  grid = n // _BLK
  return pl.pallas_call(
      _proj_body,
      grid=(grid,),
      in_specs=[
          pl.BlockSpec((_BLK, x.shape[1]), lambda i: (i, 0)),
          pl.BlockSpec(w1a.shape, lambda i: (0, 0)),
      ],
      out_specs=pl.BlockSpec((_BLK, F_PAD), lambda i: (i, 0)),
      out_shape=jax.ShapeDtypeStruct((n, F_PAD), jnp.float32),
  )(x, w1a)


def _mid_body(p1_ref, a0_ref, a1_ref, b1a_ref, w1b_ref, b1b_ref, w2a_ref,
              o_ref):
  z = p1_ref[...] + a0_ref[...] + a1_ref[...] + b1a_ref[...]
  z = jnp.maximum(z, 0.0)
  h = jnp.dot(z, w1b_ref[...], preferred_element_type=jnp.float32)
  h = jnp.maximum(h + b1b_ref[...], 0.0)
  o_ref[...] = jnp.dot(h, w2a_ref[...], preferred_element_type=jnp.float32)


def _mid(p1, a0, a1, b1a, w1b, b1b, w2a_p):
  n = p1.shape[0]
  grid = n // _BLK
  row = lambda i: (i, 0)
  rep = lambda i: (0, 0)
  return pl.pallas_call(
      _mid_body,
      grid=(grid,),
      in_specs=[
          pl.BlockSpec((_BLK, F_PAD), row),
          pl.BlockSpec((_BLK, F_PAD), row),
          pl.BlockSpec((_BLK, F_PAD), row),
          pl.BlockSpec((1, F_PAD), rep),
          pl.BlockSpec((F_PAD, F_PAD), rep),
          pl.BlockSpec((1, F_PAD), rep),
          pl.BlockSpec((F_PAD, F_PAD), rep),
      ],
      out_specs=pl.BlockSpec((_BLK, F_PAD), row),
      out_shape=jax.ShapeDtypeStruct((n, F_PAD), jnp.float32),
  )(p1, a0, a1, b1a, w1b, b1b, w2a_p)


def _tail_body(p2_ref, a0_ref, a1_ref, batch_ref, b2a_ref, w2b_ref, b2b_ref,
               wfc_ref, bfc_ref, o_ref, sums_ref, cnts_ref):
  i = pl.program_id(0)
  ng = pl.num_programs(0)

  @pl.when(i == 0)
  def _():
    sums_ref[...] = jnp.zeros_like(sums_ref)
    cnts_ref[...] = jnp.zeros_like(cnts_ref)

  z = p2_ref[...] + a0_ref[...] + a1_ref[...] + b2a_ref[...]
  z = jnp.maximum(z, 0.0)
  h = jnp.dot(z, w2b_ref[...], preferred_element_type=jnp.float32)
  h = jnp.maximum(h + b2b_ref[...], 0.0)
  g = batch_ref[...]
  onehot = (g == lax.broadcasted_iota(jnp.int32, (1, 64), 1)).astype(
      jnp.float32)
  sums_ref[...] += lax.dot_general(onehot, h, (((0,), (0,)), ((), ())),
                                   preferred_element_type=jnp.float32)
  cnts_ref[...] += jnp.sum(onehot, axis=0, keepdims=True)

  @pl.when(i == ng - 1)
  def _():
    pooled = sums_ref[...] / jnp.maximum(cnts_ref[...].reshape(64, 1), 1.0)
    logits = jnp.dot(pooled, wfc_ref[...],
                     preferred_element_type=jnp.float32) + bfc_ref[...]
    m = jnp.max(logits, axis=1, keepdims=True)
    lse = jnp.log(jnp.sum(jnp.exp(logits - m), axis=1, keepdims=True)) + m
    o_ref[...] = logits - lse


def _tail(p2, a0, a1, batch2d, b2a_p, w2b_p, b2b_p, wfc_p, bfc):
  n = p2.shape[0]
  c = wfc_p.shape[1]
  grid = n // _BLK
  row = lambda i: (i, 0)
  rep = lambda i: (0, 0)
  return pl.pallas_call(
      _tail_body,
      grid=(grid,),
      in_specs=[
          pl.BlockSpec((_BLK, F_PAD), row),
          pl.BlockSpec((_BLK, F_PAD), row),
          pl.BlockSpec((_BLK, F_PAD), row),
          pl.BlockSpec((_BLK, 1), row),
          pl.BlockSpec((1, F_PAD), rep),
          pl.BlockSpec((F_PAD, F_PAD), rep),
          pl.BlockSpec((1, F_PAD), rep),
          pl.BlockSpec((F_PAD, c), rep),
          pl.BlockSpec((1, c), rep),
      ],
      out_specs=pl.BlockSpec((64, c), rep),
      out_shape=jax.ShapeDtypeStruct((64, c), jnp.float32),
      scratch_shapes=[
          pltpu.VMEM((64, F_PAD), jnp.float32),
          pltpu.VMEM((1, 64), jnp.float32),
      ],
  )(p2, a0, a1, batch2d, b2a_p, w2b_p, b2b_p, wfc_p, bfc)


def _pad_cols(w, width):
  return jnp.pad(w, ((0, 0), (0, width - w.shape[1])))


def kernel(x, edge_index, batch, W1a, b1a, W1b, b1b, W2a, b2a, W2b, b2b,
           Wfc, bfc):
  e = edge_index.shape[1]
  k = 125
  src2d = edge_index[0].reshape(e // k, k)
  dst2d = edge_index[1].reshape(e // k, k)
  batch2d = batch.reshape(-1, 1)

  w2a_p = _pad_cols(W2a, F_PAD)
  b2a_p = jnp.pad(b2a, (0, F_PAD - b2a.shape[0])).reshape(1, F_PAD)
  w2b_p = _pad_cols(jnp.pad(W2b, ((0, F_PAD - W2b.shape[0]), (0, 0))), F_PAD)
  b2b_p = jnp.pad(b2b, (0, F_PAD - b2b.shape[0])).reshape(1, F_PAD)
  wfc_p = jnp.pad(Wfc, ((0, F_PAD - Wfc.shape[0]), (0, 0)))
  bfc2d = bfc.reshape(1, -1)
  b1a2d = b1a.reshape(1, -1)
  b1b2d = b1b.reshape(1, -1)

  p1 = _proj(x, W1a)
  parts1 = _sc_segment_sum(p1, src2d, dst2d)
  p2 = _mid(p1, parts1[0], parts1[1], b1a2d, W1b, b1b2d, w2a_p)
  parts2 = _sc_segment_sum(p2, src2d, dst2d)
  return _tail(p2, parts2[0], parts2[1], batch2d, b2a_p, w2b_p, b2b_p,
               wfc_p, bfc2d)


# speedup vs baseline: 15.6784x; 1.3342x over previous
import functools

import jax
import jax.numpy as jnp
from jax import lax
from jax.experimental import pallas as pl
from jax.experimental.pallas import tpu as pltpu, tpu_sc as plsc

F_PAD = 16


def _sc_segment_sum(p, src2d, dst2d):
  n = p.shape[0]
  nchunk, k = src2d.shape
  nc, ns = 2, 16
  cpt = nchunk // (nc * ns)
  rows_a = 632
  rows_b = n - 15 * rows_a

  mesh = plsc.VectorSubcoreMesh(core_axis_name="c", subcore_axis_name="s")

  @functools.partial(
      pl.kernel,
      out_type=jax.ShapeDtypeStruct((nc, n, F_PAD), jnp.float32),
      mesh=mesh,
      compiler_params=pltpu.CompilerParams(use_tc_tiling_on_sc=False),
      scratch_types=[
          pltpu.VMEM((cpt, k), jnp.int32),
          pltpu.VMEM((cpt, k), jnp.int32),
          pltpu.VMEM((k, F_PAD), jnp.float32),
          pltpu.VMEM((k, F_PAD), jnp.float32),
          pltpu.VMEM((rows_a, F_PAD), jnp.float32),
          pltpu.VMEM_SHARED((n, F_PAD), jnp.float32),
          pltpu.SemaphoreType.DMA,
          pltpu.SemaphoreType.DMA,
      ],
  )
  def k_fn(p_hbm, src_hbm, dst_hbm, out_hbm, sidx, didx, rows0, rows1, zbuf,
           acc, sem0, sem1):
    c = lax.axis_index("c")
    s = lax.axis_index("s")
    tile = c * ns + s

    def zrow(i, carry):
      zbuf[i, :] = jnp.zeros((F_PAD,), jnp.float32)
      return carry

    lax.fori_loop(0, rows_a, zrow, 0)

    @pl.when(s < ns - 1)
    def _():
      pltpu.sync_copy(zbuf.at[pl.ds(0, rows_a)],
                      acc.at[pl.ds(s * rows_a, rows_a)])

    @pl.when(s == ns - 1)
    def _():
      pltpu.sync_copy(zbuf.at[pl.ds(0, rows_b)],
                      acc.at[pl.ds((ns - 1) * rows_a, rows_b)])

    pltpu.sync_copy(src_hbm.at[pl.ds(tile * cpt, cpt)], sidx)
    pltpu.sync_copy(dst_hbm.at[pl.ds(tile * cpt, cpt)], didx)
    plsc.subcore_barrier()

    pltpu.async_copy(p_hbm.at[sidx.at[0]], rows0, sem0)
    pltpu.async_copy(p_hbm.at[sidx.at[1]], rows1, sem1)

    def chunk(i, carry):
      j = 2 * i
      pltpu.make_async_copy(p_hbm.at[sidx.at[j]], rows0, sem0).wait()
      pltpu.sync_copy(rows0, acc.at[didx.at[j]], add=True)

      @pl.when(j + 2 < cpt)
      def _():
        pltpu.async_copy(p_hbm.at[sidx.at[j + 2]], rows0, sem0)

      pltpu.make_async_copy(p_hbm.at[sidx.at[j + 1]], rows1, sem1).wait()
      pltpu.sync_copy(rows1, acc.at[didx.at[j + 1]], add=True)

      @pl.when(j + 3 < cpt)
      def _():
        pltpu.async_copy(p_hbm.at[sidx.at[j + 3]], rows1, sem1)

      return carry

    lax.fori_loop(0, cpt // 2, chunk, 0)
    plsc.subcore_barrier()

    @pl.when(s < ns - 1)
    def _():
      pltpu.sync_copy(acc.at[pl.ds(s * rows_a, rows_a)],
                      out_hbm.at[c, pl.ds(s * rows_a, rows_a)])

    @pl.when(s == ns - 1)
    def _():
      pltpu.sync_copy(acc.at[pl.ds((ns - 1) * rows_a, rows_b)],
                      out_hbm.at[c, pl.ds((ns - 1) * rows_a, rows_b)])

  return k_fn(p, src2d, dst2d)


_BLK = 1000


def _proj_body(x_ref, w_ref, o_ref):
  o_ref[...] = jnp.dot(x_ref[...], w_ref[...],
                       preferred_element_type=jnp.float32)


def _proj(x, w1a):
  n = x.shape[0]
  grid = n // _BLK
  return pl.pallas_call(
      _proj_body,
      grid=(grid,),
      in_specs=[
          pl.BlockSpec((_BLK, x.shape[1]), lambda i: (i, 0)),
          pl.BlockSpec(w1a.shape, lambda i: (0, 0)),
      ],
      out_specs=pl.BlockSpec((_BLK, F_PAD), lambda i: (i, 0)),
      out_shape=jax.ShapeDtypeStruct((n, F_PAD), jnp.float32),
  )(x, w1a)


def _mid_body(p1_ref, a0_ref, a1_ref, b1a_ref, w1b_ref, b1b_ref, w2a_ref,
              o_ref):
  z = p1_ref[...] + a0_ref[...] + a1_ref[...] + b1a_ref[...]
  z = jnp.maximum(z, 0.0)
  h = jnp.dot(z, w1b_ref[...], preferred_element_type=jnp.float32)
  h = jnp.maximum(h + b1b_ref[...], 0.0)
  o_ref[...] = jnp.dot(h, w2a_ref[...], preferred_element_type=jnp.float32)


def _mid(p1, a0, a1, b1a, w1b, b1b, w2a_p):
  n = p1.shape[0]
  grid = n // _BLK
  row = lambda i: (i, 0)
  rep = lambda i: (0, 0)
  return pl.pallas_call(
      _mid_body,
      grid=(grid,),
      in_specs=[
          pl.BlockSpec((_BLK, F_PAD), row),
          pl.BlockSpec((_BLK, F_PAD), row),
          pl.BlockSpec((_BLK, F_PAD), row),
          pl.BlockSpec((1, F_PAD), rep),
          pl.BlockSpec((F_PAD, F_PAD), rep),
          pl.BlockSpec((1, F_PAD), rep),
          pl.BlockSpec((F_PAD, F_PAD), rep),
      ],
      out_specs=pl.BlockSpec((_BLK, F_PAD), row),
      out_shape=jax.ShapeDtypeStruct((n, F_PAD), jnp.float32),
  )(p1, a0, a1, b1a, w1b, b1b, w2a_p)


def _tail_body(p2_ref, a0_ref, a1_ref, batch_ref, b2a_ref, w2b_ref, b2b_ref,
               wfc_ref, bfc_ref, o_ref, sums_ref, cnts_ref):
  i = pl.program_id(0)
  ng = pl.num_programs(0)

  @pl.when(i == 0)
  def _():
    sums_ref[...] = jnp.zeros_like(sums_ref)
    cnts_ref[...] = jnp.zeros_like(cnts_ref)

  z = p2_ref[...] + a0_ref[...] + a1_ref[...] + b2a_ref[...]
  z = jnp.maximum(z, 0.0)
  h = jnp.dot(z, w2b_ref[...], preferred_element_type=jnp.float32)
  h = jnp.maximum(h + b2b_ref[...], 0.0)
  g = batch_ref[...]
  onehot = (g == lax.broadcasted_iota(jnp.int32, (1, 64), 1)).astype(
      jnp.float32)
  sums_ref[...] += lax.dot_general(onehot, h, (((0,), (0,)), ((), ())),
                                   preferred_element_type=jnp.float32)
  cnts_ref[...] += jnp.sum(onehot, axis=0, keepdims=True)

  @pl.when(i == ng - 1)
  def _():
    pooled = sums_ref[...] / jnp.maximum(cnts_ref[...].reshape(64, 1), 1.0)
    logits = jnp.dot(pooled, wfc_ref[...],
                     preferred_element_type=jnp.float32) + bfc_ref[...]
    m = jnp.max(logits, axis=1, keepdims=True)
    lse = jnp.log(jnp.sum(jnp.exp(logits - m), axis=1, keepdims=True)) + m
    o_ref[...] = logits - lse


def _tail(p2, a0, a1, batch2d, b2a_p, w2b_p, b2b_p, wfc_p, bfc):
  n = p2.shape[0]
  c = wfc_p.shape[1]
  grid = n // _BLK
  row = lambda i: (i, 0)
  rep = lambda i: (0, 0)
  return pl.pallas_call(
      _tail_body,
      grid=(grid,),
      in_specs=[
          pl.BlockSpec((_BLK, F_PAD), row),
          pl.BlockSpec((_BLK, F_PAD), row),
          pl.BlockSpec((_BLK, F_PAD), row),
          pl.BlockSpec((_BLK, 1), row),
          pl.BlockSpec((1, F_PAD), rep),
          pl.BlockSpec((F_PAD, F_PAD), rep),
          pl.BlockSpec((1, F_PAD), rep),
          pl.BlockSpec((F_PAD, c), rep),
          pl.BlockSpec((1, c), rep),
      ],
      out_specs=pl.BlockSpec((64, c), rep),
      out_shape=jax.ShapeDtypeStruct((64, c), jnp.float32),
      scratch_shapes=[
          pltpu.VMEM((64, F_PAD), jnp.float32),
          pltpu.VMEM((1, 64), jnp.float32),
      ],
  )(p2, a0, a1, batch2d, b2a_p, w2b_p, b2b_p, wfc_p, bfc)


def _pad_cols(w, width):
  return jnp.pad(w, ((0, 0), (0, width - w.shape[1])))


def kernel(x, edge_index, batch, W1a, b1a, W1b, b1b, W2a, b2a, W2b, b2b,
           Wfc, bfc):
  e = edge_index.shape[1]
  k = 125
  src2d = edge_index[0].reshape(e // k, k)
  dst2d = edge_index[1].reshape(e // k, k)
  batch2d = batch.reshape(-1, 1)

  w2a_p = _pad_cols(W2a, F_PAD)
  b2a_p = jnp.pad(b2a, (0, F_PAD - b2a.shape[0])).reshape(1, F_PAD)
  w2b_p = _pad_cols(jnp.pad(W2b, ((0, F_PAD - W2b.shape[0]), (0, 0))), F_PAD)
  b2b_p = jnp.pad(b2b, (0, F_PAD - b2b.shape[0])).reshape(1, F_PAD)
  wfc_p = jnp.pad(Wfc, ((0, F_PAD - Wfc.shape[0]), (0, 0)))
  bfc2d = bfc.reshape(1, -1)
  b1a2d = b1a.reshape(1, -1)
  b1b2d = b1b.reshape(1, -1)

  p1 = _proj(x, W1a)
  parts1 = _sc_segment_sum(p1, src2d, dst2d)
  p2 = _mid(p1, parts1[0], parts1[1], b1a2d, W1b, b1b2d, w2a_p)
  parts2 = _sc_segment_sum(p2, src2d, dst2d)
  return _tail(p2, parts2[0], parts2[1], batch2d, b2a_p, w2b_p, b2b_p,
               wfc_p, bfc2d)


# speedup vs baseline: 22.1329x; 1.4117x over previous
import functools

import jax
import jax.numpy as jnp
from jax import lax
from jax.experimental import pallas as pl
from jax.experimental.pallas import tpu as pltpu, tpu_sc as plsc

F_PAD = 16
PACK = 8


def _sc_segment_sum(p, edge3d):
  n = p.shape[0]
  _, nchunk, k = edge3d.shape
  nc, ns = 2, 16
  cpt = nchunk // (nc * ns)
  rows_a = 632
  rows_b = n - 15 * rows_a

  mesh = plsc.VectorSubcoreMesh(core_axis_name="c", subcore_axis_name="s")

  @functools.partial(
      pl.kernel,
      out_type=jax.ShapeDtypeStruct((nc, n, F_PAD), jnp.float32),
      mesh=mesh,
      compiler_params=pltpu.CompilerParams(use_tc_tiling_on_sc=False),
      scratch_types=[
          pltpu.VMEM((cpt, k), jnp.int32),
          pltpu.VMEM((cpt, k), jnp.int32),
          pltpu.VMEM((k, F_PAD), jnp.float32),
          pltpu.VMEM((k, F_PAD), jnp.float32),
          pltpu.VMEM((rows_a, F_PAD), jnp.float32),
          pltpu.VMEM_SHARED((n, F_PAD), jnp.float32),
          pltpu.SemaphoreType.DMA,
          pltpu.SemaphoreType.DMA,
      ],
  )
  def k_fn(p_hbm, edge_hbm, out_hbm, sidx, didx, rows0, rows1, zbuf,
           acc, sem0, sem1):
    c = lax.axis_index("c")
    s = lax.axis_index("s")
    tile = c * ns + s

    def zrow(i, carry):
      zbuf[i, :] = jnp.zeros((F_PAD,), jnp.float32)
      return carry

    lax.fori_loop(0, rows_a, zrow, 0)

    @pl.when(s < ns - 1)
    def _():
      pltpu.sync_copy(zbuf.at[pl.ds(0, rows_a)],
                      acc.at[pl.ds(s * rows_a, rows_a)])

    @pl.when(s == ns - 1)
    def _():
      pltpu.sync_copy(zbuf.at[pl.ds(0, rows_b)],
                      acc.at[pl.ds((ns - 1) * rows_a, rows_b)])

    pltpu.sync_copy(edge_hbm.at[0, pl.ds(tile * cpt, cpt)], sidx)
    pltpu.sync_copy(edge_hbm.at[1, pl.ds(tile * cpt, cpt)], didx)
    plsc.subcore_barrier()

    pltpu.async_copy(p_hbm.at[sidx.at[0]], rows0, sem0)
    pltpu.async_copy(p_hbm.at[sidx.at[1]], rows1, sem1)

    def chunk(i, carry):
      j = 2 * i
      pltpu.make_async_copy(p_hbm.at[sidx.at[j]], rows0, sem0).wait()
      pltpu.sync_copy(rows0, acc.at[didx.at[j]], add=True)

      @pl.when(j + 2 < cpt)
      def _():
        pltpu.async_copy(p_hbm.at[sidx.at[j + 2]], rows0, sem0)

      pltpu.make_async_copy(p_hbm.at[sidx.at[j + 1]], rows1, sem1).wait()
      pltpu.sync_copy(rows1, acc.at[didx.at[j + 1]], add=True)

      @pl.when(j + 3 < cpt)
      def _():
        pltpu.async_copy(p_hbm.at[sidx.at[j + 3]], rows1, sem1)

      return carry

    lax.fori_loop(0, cpt // 2, chunk, 0)
    plsc.subcore_barrier()

    @pl.when(s < ns - 1)
    def _():
      pltpu.sync_copy(acc.at[pl.ds(s * rows_a, rows_a)],
                      out_hbm.at[c, pl.ds(s * rows_a, rows_a)])

    @pl.when(s == ns - 1)
    def _():
      pltpu.sync_copy(acc.at[pl.ds((ns - 1) * rows_a, rows_b)],
                      out_hbm.at[c, pl.ds((ns - 1) * rows_a, rows_b)])

  return k_fn(p, edge3d)


def _proj_body(x_ref, w_ref, o_ref):
  o_ref[...] = jnp.dot(x_ref[...], w_ref[...],
                       preferred_element_type=jnp.float32)


def _proj(x8, w1a_bd):
  r = x8.shape[0]
  grid = 1
  blk = r // grid
  return pl.pallas_call(
      _proj_body,
      grid=(grid,),
      in_specs=[
          pl.BlockSpec((blk, x8.shape[1]), lambda i: (i, 0)),
          pl.BlockSpec(w1a_bd.shape, lambda i: (0, 0)),
      ],
      out_specs=pl.BlockSpec((blk, 128), lambda i: (i, 0)),
      out_shape=jax.ShapeDtypeStruct((r, 128), jnp.float32),
  )(x8, w1a_bd)


def _mid_body(p1_ref, parts_ref, b1a_ref, w1b_ref, b1b_ref, w2a_ref, o_ref):
  z = p1_ref[...] + parts_ref[0] + parts_ref[1] + b1a_ref[...]
  z = jnp.maximum(z, 0.0)
  h = jnp.dot(z, w1b_ref[...], preferred_element_type=jnp.float32)
  h = jnp.maximum(h + b1b_ref[...], 0.0)
  o_ref[...] = jnp.dot(h, w2a_ref[...], preferred_element_type=jnp.float32)


def _mid(p1v, partsv, b1a_t, w1b_bd, b1b_t, w2a_bd):
  r = p1v.shape[0]
  grid = 1
  blk = r // grid
  row = lambda i: (i, 0)
  rep = lambda i: (0, 0)
  return pl.pallas_call(
      _mid_body,
      grid=(grid,),
      in_specs=[
          pl.BlockSpec((blk, 128), row),
          pl.BlockSpec((2, blk, 128), lambda i: (0, i, 0)),
          pl.BlockSpec((1, 128), rep),
          pl.BlockSpec((128, 128), rep),
          pl.BlockSpec((1, 128), rep),
          pl.BlockSpec((128, 128), rep),
      ],
      out_specs=pl.BlockSpec((blk, 128), row),
      out_shape=jax.ShapeDtypeStruct((r, 128), jnp.float32),
  )(p1v, partsv, b1a_t, w1b_bd, b1b_t, w2a_bd)


def _tail_body(p2_ref, parts_ref, batch_ref, b2a_ref, w2b_ref, b2b_ref,
               wfc_ref, bfc_ref, o_ref):
  z = p2_ref[...] + parts_ref[0] + parts_ref[1] + b2a_ref[...]
  z = jnp.maximum(z, 0.0)
  h = jnp.dot(z, w2b_ref[...], preferred_element_type=jnp.float32)
  h = jnp.maximum(h + b2b_ref[...], 0.0)

  r = h.shape[0]
  giota = lax.broadcasted_iota(jnp.int32, (1, 64), 1)
  ones = jnp.ones((r, 1), jnp.float32)
  sums = jnp.zeros((64, F_PAD), jnp.float32)
  cnts = jnp.zeros((64, 1), jnp.float32)
  for s in range(PACK):
    oh = (batch_ref[:, s:s + 1] == giota).astype(jnp.float32)
    sums += lax.dot_general(oh, h[:, s * F_PAD:(s + 1) * F_PAD],
                            (((0,), (0,)), ((), ())),
                            preferred_element_type=jnp.float32)
    cnts += lax.dot_general(oh, ones, (((0,), (0,)), ((), ())),
                            preferred_element_type=jnp.float32)

  pooled = sums / jnp.maximum(cnts, 1.0)
  logits = jnp.dot(pooled, wfc_ref[...],
                   preferred_element_type=jnp.float32) + bfc_ref[...]
  m = jnp.max(logits, axis=1, keepdims=True)
  lse = jnp.log(jnp.sum(jnp.exp(logits - m), axis=1, keepdims=True)) + m
  o_ref[...] = logits - lse


def _tail(p2v, partsv, batchv, b2a_t, w2b_bd, b2b_t, wfc_p, bfc2d):
  r = p2v.shape[0]
  c = wfc_p.shape[1]
  full = lambda i: (0, 0)
  return pl.pallas_call(
      _tail_body,
      grid=(1,),
      in_specs=[
          pl.BlockSpec((r, 128), full),
          pl.BlockSpec((2, r, 128), lambda i: (0, 0, 0)),
          pl.BlockSpec((r, PACK), full),
          pl.BlockSpec((1, 128), full),
          pl.BlockSpec((128, 128), full),
          pl.BlockSpec((1, 128), full),
          pl.BlockSpec((F_PAD, c), full),
          pl.BlockSpec((1, c), full),
      ],
      out_specs=pl.BlockSpec((64, c), full),
      out_shape=jax.ShapeDtypeStruct((64, c), jnp.float32),
  )(p2v, partsv, batchv, b2a_t, w2b_bd, b2b_t, wfc_p, bfc2d)


def kernel(x, edge_index, batch, W1a, b1a, W1b, b1b, W2a, b2a, W2b, b2b,
           Wfc, bfc):
  n = x.shape[0]
  e = edge_index.shape[1]
  k = 125
  r = n // PACK
  edge3d = edge_index.reshape(2, e // k, k)
  batchv = batch.reshape(r, PACK)
  x8 = x.reshape(r, PACK * x.shape[1])

  eye = jnp.eye(PACK, dtype=jnp.float32)
  w1a_bd = jnp.kron(eye, W1a)
  w1b_bd = jnp.kron(eye, W1b)
  w2a_p = jnp.pad(W2a, ((0, 0), (0, F_PAD - W2a.shape[1])))
  w2a_bd = jnp.kron(eye, w2a_p)
  w2b_p = jnp.pad(W2b, ((0, F_PAD - W2b.shape[0]),
                        (0, F_PAD - W2b.shape[1])))
  w2b_bd = jnp.kron(eye, w2b_p)
  wfc_p = jnp.pad(Wfc, ((0, F_PAD - Wfc.shape[0]), (0, 0)))
  b1a_t = jnp.tile(b1a, PACK).reshape(1, 128)
  b1b_t = jnp.tile(b1b, PACK).reshape(1, 128)
  b2a_t = jnp.tile(jnp.pad(b2a, (0, F_PAD - b2a.shape[0])),
                   PACK).reshape(1, 128)
  b2b_t = jnp.tile(jnp.pad(b2b, (0, F_PAD - b2b.shape[0])),
                   PACK).reshape(1, 128)
  bfc2d = bfc.reshape(1, -1)

  p1v = _proj(x8, w1a_bd)
  parts1 = _sc_segment_sum(p1v.reshape(n, F_PAD), edge3d)
  p2v = _mid(p1v, parts1.reshape(2, r, 128), b1a_t, w1b_bd, b1b_t, w2a_bd)
  parts2 = _sc_segment_sum(p2v.reshape(n, F_PAD), edge3d)
  return _tail(p2v, parts2.reshape(2, r, 128), batchv, b2a_t, w2b_bd, b2b_t,
               wfc_p, bfc2d)


# speedup vs baseline: 31.2384x; 1.4114x over previous
import functools

import jax
import jax.numpy as jnp
from jax import lax
from jax.experimental import pallas as pl
from jax.experimental.pallas import tpu as pltpu, tpu_sc as plsc

F_PAD = 16
PACK = 8
NBUF = 8


def _sc_segment_sum(p, edge3d):
  n = p.shape[0]
  _, nchunk, k = edge3d.shape
  nc, ns = 2, 16
  cpt = nchunk // (nc * ns)
  rows_a = 632
  rows_b = n - 15 * rows_a

  mesh = plsc.VectorSubcoreMesh(core_axis_name="c", subcore_axis_name="s")

  @functools.partial(
      pl.kernel,
      out_type=jax.ShapeDtypeStruct((nc, n, F_PAD), jnp.float32),
      mesh=mesh,
      compiler_params=pltpu.CompilerParams(use_tc_tiling_on_sc=False),
      scratch_types=[
          pltpu.VMEM((cpt, k), jnp.int32),
          pltpu.VMEM((cpt, k), jnp.int32),
          pltpu.VMEM((rows_a, F_PAD), jnp.float32),
          pltpu.VMEM_SHARED((n, F_PAD), jnp.float32),
      ] + [pltpu.VMEM((k, F_PAD), jnp.float32) for _ in range(NBUF)]
        + [pltpu.SemaphoreType.DMA for _ in range(2 * NBUF)],
  )
  def k_fn(p_hbm, edge_hbm, out_hbm, sidx, didx, zbuf, acc, *bufsem):
    rows = bufsem[:NBUF]
    gsem = bufsem[NBUF:2 * NBUF]
    ssem = bufsem[2 * NBUF:]
    c = lax.axis_index("c")
    s = lax.axis_index("s")
    tile = c * ns + s

    def zrow(i, carry):
      zbuf[i, :] = jnp.zeros((F_PAD,), jnp.float32)
      return carry

    lax.fori_loop(0, rows_a, zrow, 0)

    @pl.when(s < ns - 1)
    def _():
      pltpu.sync_copy(zbuf.at[pl.ds(0, rows_a)],
                      acc.at[pl.ds(s * rows_a, rows_a)])

    @pl.when(s == ns - 1)
    def _():
      pltpu.sync_copy(zbuf.at[pl.ds(0, rows_b)],
                      acc.at[pl.ds((ns - 1) * rows_a, rows_b)])

    pltpu.sync_copy(edge_hbm.at[0, pl.ds(tile * cpt, cpt)], sidx)
    pltpu.sync_copy(edge_hbm.at[1, pl.ds(tile * cpt, cpt)], didx)
    plsc.subcore_barrier()

    for b in range(NBUF):
      pltpu.async_copy(p_hbm.at[sidx.at[b]], rows[b], gsem[b])

    def chunk(i, carry):
      j = NBUF * i
      for b in range(NBUF):
        pltpu.make_async_copy(p_hbm.at[sidx.at[j + b]], rows[b],
                              gsem[b]).wait()
        pltpu.async_copy(rows[b], acc.at[didx.at[j + b]], ssem[b], add=True)
      for b in range(NBUF):
        @pl.when(j + b + NBUF < cpt)
        def _(b=b):
          pltpu.make_async_copy(rows[b], acc.at[didx.at[j + b]],
                                ssem[b]).wait()
          pltpu.async_copy(p_hbm.at[sidx.at[j + b + NBUF]], rows[b], gsem[b])

      return carry

    lax.fori_loop(0, cpt // NBUF, chunk, 0)
    for b in range(NBUF):
      pltpu.make_async_copy(rows[b], acc.at[didx.at[cpt - NBUF + b]],
                            ssem[b]).wait()
    plsc.subcore_barrier()

    @pl.when(s < ns - 1)
    def _():
      pltpu.sync_copy(acc.at[pl.ds(s * rows_a, rows_a)],
                      out_hbm.at[c, pl.ds(s * rows_a, rows_a)])

    @pl.when(s == ns - 1)
    def _():
      pltpu.sync_copy(acc.at[pl.ds((ns - 1) * rows_a, rows_b)],
                      out_hbm.at[c, pl.ds((ns - 1) * rows_a, rows_b)])

  return k_fn(p, edge3d)


def _proj_body(x_ref, w_ref, o_ref):
  o_ref[...] = jnp.dot(x_ref[...], w_ref[...],
                       preferred_element_type=jnp.float32)


def _proj(x8, w1a_bd):
  r = x8.shape[0]
  grid = 1
  blk = r // grid
  return pl.pallas_call(
      _proj_body,
      grid=(grid,),
      in_specs=[
          pl.BlockSpec((blk, x8.shape[1]), lambda i: (i, 0)),
          pl.BlockSpec(w1a_bd.shape, lambda i: (0, 0)),
      ],
      out_specs=pl.BlockSpec((blk, 128), lambda i: (i, 0)),
      out_shape=jax.ShapeDtypeStruct((r, 128), jnp.float32),
  )(x8, w1a_bd)


def _mid_body(p1_ref, parts_ref, b1a_ref, w1b_ref, b1b_ref, w2a_ref, o_ref):
  z = p1_ref[...] + parts_ref[0] + parts_ref[1] + b1a_ref[...]
  z = jnp.maximum(z, 0.0)
  h = jnp.dot(z, w1b_ref[...], preferred_element_type=jnp.float32)
  h = jnp.maximum(h + b1b_ref[...], 0.0)
  o_ref[...] = jnp.dot(h, w2a_ref[...], preferred_element_type=jnp.float32)


def _mid(p1v, partsv, b1a_t, w1b_bd, b1b_t, w2a_bd):
  r = p1v.shape[0]
  grid = 1
  blk = r // grid
  row = lambda i: (i, 0)
  rep = lambda i: (0, 0)
  return pl.pallas_call(
      _mid_body,
      grid=(grid,),
      in_specs=[
          pl.BlockSpec((blk, 128), row),
          pl.BlockSpec((2, blk, 128), lambda i: (0, i, 0)),
          pl.BlockSpec((1, 128), rep),
          pl.BlockSpec((128, 128), rep),
          pl.BlockSpec((1, 128), rep),
          pl.BlockSpec((128, 128), rep),
      ],
      out_specs=pl.BlockSpec((blk, 128), row),
      out_shape=jax.ShapeDtypeStruct((r, 128), jnp.float32),
  )(p1v, partsv, b1a_t, w1b_bd, b1b_t, w2a_bd)


def _tail_body(p2_ref, parts_ref, batch_ref, b2a_ref, w2b_ref, b2b_ref,
               wfc_ref, bfc_ref, o_ref):
  z = p2_ref[...] + parts_ref[0] + parts_ref[1] + b2a_ref[...]
  z = jnp.maximum(z, 0.0)
  h = jnp.dot(z, w2b_ref[...], preferred_element_type=jnp.float32)
  h = jnp.maximum(h + b2b_ref[...], 0.0)

  r = h.shape[0]
  giota = lax.broadcasted_iota(jnp.int32, (1, 64), 1)
  ones = jnp.ones((r, 1), jnp.float32)
  sums = jnp.zeros((64, F_PAD), jnp.float32)
  cnts = jnp.zeros((64, 1), jnp.float32)
  for s in range(PACK):
    oh = (batch_ref[:, s:s + 1] == giota).astype(jnp.float32)
    sums += lax.dot_general(oh, h[:, s * F_PAD:(s + 1) * F_PAD],
                            (((0,), (0,)), ((), ())),
                            preferred_element_type=jnp.float32)
    cnts += lax.dot_general(oh, ones, (((0,), (0,)), ((), ())),
                            preferred_element_type=jnp.float32)

  pooled = sums / jnp.maximum(cnts, 1.0)
  logits = jnp.dot(pooled, wfc_ref[...],
                   preferred_element_type=jnp.float32) + bfc_ref[...]
  m = jnp.max(logits, axis=1, keepdims=True)
  lse = jnp.log(jnp.sum(jnp.exp(logits - m), axis=1, keepdims=True)) + m
  o_ref[...] = logits - lse


def _tail(p2v, partsv, batchv, b2a_t, w2b_bd, b2b_t, wfc_p, bfc2d):
  r = p2v.shape[0]
  c = wfc_p.shape[1]
  full = lambda i: (0, 0)
  return pl.pallas_call(
      _tail_body,
      grid=(1,),
      in_specs=[
          pl.BlockSpec((r, 128), full),
          pl.BlockSpec((2, r, 128), lambda i: (0, 0, 0)),
          pl.BlockSpec((r, PACK), full),
          pl.BlockSpec((1, 128), full),
          pl.BlockSpec((128, 128), full),
          pl.BlockSpec((1, 128), full),
          pl.BlockSpec((F_PAD, c), full),
          pl.BlockSpec((1, c), full),
      ],
      out_specs=pl.BlockSpec((64, c), full),
      out_shape=jax.ShapeDtypeStruct((64, c), jnp.float32),
  )(p2v, partsv, batchv, b2a_t, w2b_bd, b2b_t, wfc_p, bfc2d)


def kernel(x, edge_index, batch, W1a, b1a, W1b, b1b, W2a, b2a, W2b, b2b,
           Wfc, bfc):
  n = x.shape[0]
  e = edge_index.shape[1]
  k = 125
  r = n // PACK
  edge3d = edge_index.reshape(2, e // k, k)
  batchv = batch.reshape(r, PACK)
  x8 = x.reshape(r, PACK * x.shape[1])

  eye = jnp.eye(PACK, dtype=jnp.float32)
  w1a_bd = jnp.kron(eye, W1a)
  w1b_bd = jnp.kron(eye, W1b)
  w2a_p = jnp.pad(W2a, ((0, 0), (0, F_PAD - W2a.shape[1])))
  w2a_bd = jnp.kron(eye, w2a_p)
  w2b_p = jnp.pad(W2b, ((0, F_PAD - W2b.shape[0]),
                        (0, F_PAD - W2b.shape[1])))
  w2b_bd = jnp.kron(eye, w2b_p)
  wfc_p = jnp.pad(Wfc, ((0, F_PAD - Wfc.shape[0]), (0, 0)))
  b1a_t = jnp.tile(b1a, PACK).reshape(1, 128)
  b1b_t = jnp.tile(b1b, PACK).reshape(1, 128)
  b2a_t = jnp.tile(jnp.pad(b2a, (0, F_PAD - b2a.shape[0])),
                   PACK).reshape(1, 128)
  b2b_t = jnp.tile(jnp.pad(b2b, (0, F_PAD - b2b.shape[0])),
                   PACK).reshape(1, 128)
  bfc2d = bfc.reshape(1, -1)

  p1v = _proj(x8, w1a_bd)
  parts1 = _sc_segment_sum(p1v.reshape(n, F_PAD), edge3d)
  p2v = _mid(p1v, parts1.reshape(2, r, 128), b1a_t, w1b_bd, b1b_t, w2a_bd)
  parts2 = _sc_segment_sum(p2v.reshape(n, F_PAD), edge3d)
  return _tail(p2v, parts2.reshape(2, r, 128), batchv, b2a_t, w2b_bd, b2b_t,
               wfc_p, bfc2d)


# speedup vs baseline: 36.0750x; 1.1548x over previous
import functools

import jax
import jax.numpy as jnp
from jax import lax
from jax.experimental import pallas as pl
from jax.experimental.pallas import tpu as pltpu, tpu_sc as plsc

F_PAD = 16
PACK = 8
NBUF = 8


def _sc_segment_sum(p, edge3d):
  n = p.shape[0]
  _, nchunk, k = edge3d.shape
  nc, ns = 2, 16
  cpt = nchunk // (nc * ns)
  rows_a = 632
  rows_b = n - 15 * rows_a

  mesh = plsc.VectorSubcoreMesh(core_axis_name="c", subcore_axis_name="s")

  @functools.partial(
      pl.kernel,
      out_type=jax.ShapeDtypeStruct((nc, n, F_PAD), jnp.float32),
      mesh=mesh,
      compiler_params=pltpu.CompilerParams(use_tc_tiling_on_sc=False),
      scratch_types=[
          pltpu.VMEM((cpt, k), jnp.int32),
          pltpu.VMEM((cpt, k), jnp.int32),
          pltpu.VMEM((rows_a, F_PAD), jnp.float32),
          pltpu.VMEM_SHARED((n, F_PAD), jnp.float32),
      ] + [pltpu.VMEM((k, F_PAD), jnp.float32) for _ in range(NBUF)]
        + [pltpu.SemaphoreType.DMA for _ in range(2 * NBUF)],
  )
  def k_fn(p_hbm, edge_hbm, out_hbm, sidx, didx, zbuf, acc, *bufsem):
    rows = bufsem[:NBUF]
    gsem = bufsem[NBUF:2 * NBUF]
    ssem = bufsem[2 * NBUF:]
    c = lax.axis_index("c")
    s = lax.axis_index("s")
    tile = c * ns + s

    pltpu.async_copy(edge_hbm.at[0, pl.ds(tile * cpt, cpt)], sidx, gsem[0])
    pltpu.async_copy(edge_hbm.at[1, pl.ds(tile * cpt, cpt)], didx, gsem[1])

    @pl.when(c == 0)
    def _():
      @pl.when(s < ns - 1)
      def _():
        pltpu.sync_copy(p_hbm.at[pl.ds(s * rows_a, rows_a)],
                        acc.at[pl.ds(s * rows_a, rows_a)])

      @pl.when(s == ns - 1)
      def _():
        pltpu.sync_copy(p_hbm.at[pl.ds((ns - 1) * rows_a, rows_b)],
                        acc.at[pl.ds((ns - 1) * rows_a, rows_b)])

    @pl.when(c == 1)
    def _():
      def zrow(i, carry):
        for u in range(8):
          zbuf[8 * i + u, :] = jnp.zeros((F_PAD,), jnp.float32)
        return carry

      lax.fori_loop(0, rows_a // 8, zrow, 0)

      @pl.when(s < ns - 1)
      def _():
        pltpu.sync_copy(zbuf.at[pl.ds(0, rows_a)],
                        acc.at[pl.ds(s * rows_a, rows_a)])

      @pl.when(s == ns - 1)
      def _():
        pltpu.sync_copy(zbuf.at[pl.ds(0, rows_b)],
                        acc.at[pl.ds((ns - 1) * rows_a, rows_b)])

    pltpu.make_async_copy(edge_hbm.at[0, pl.ds(tile * cpt, cpt)], sidx,
                          gsem[0]).wait()
    pltpu.make_async_copy(edge_hbm.at[1, pl.ds(tile * cpt, cpt)], didx,
                          gsem[1]).wait()
    plsc.subcore_barrier()

    for b in range(NBUF):
      pltpu.async_copy(p_hbm.at[sidx.at[b]], rows[b], gsem[b])

    def chunk(i, carry):
      j = NBUF * i
      for b in range(NBUF):
        pltpu.make_async_copy(p_hbm.at[sidx.at[j + b]], rows[b],
                              gsem[b]).wait()
        pltpu.async_copy(rows[b], acc.at[didx.at[j + b]], ssem[b], add=True)
      for b in range(NBUF):
        @pl.when(j + b + NBUF < cpt)
        def _(b=b):
          pltpu.make_async_copy(rows[b], acc.at[didx.at[j + b]],
                                ssem[b]).wait()
          pltpu.async_copy(p_hbm.at[sidx.at[j + b + NBUF]], rows[b], gsem[b])

      return carry

    lax.fori_loop(0, cpt // NBUF, chunk, 0)
    for b in range(NBUF):
      pltpu.make_async_copy(rows[b], acc.at[didx.at[cpt - NBUF + b]],
                            ssem[b]).wait()
    plsc.subcore_barrier()

    @pl.when(s < ns - 1)
    def _():
      pltpu.sync_copy(acc.at[pl.ds(s * rows_a, rows_a)],
                      out_hbm.at[c, pl.ds(s * rows_a, rows_a)])

    @pl.when(s == ns - 1)
    def _():
      pltpu.sync_copy(acc.at[pl.ds((ns - 1) * rows_a, rows_b)],
                      out_hbm.at[c, pl.ds((ns - 1) * rows_a, rows_b)])

  return k_fn(p, edge3d)


def _proj_body(x_ref, w_ref, o_ref):
  v = x_ref[...]
  p = lax.dot_general(v, w_ref[...], (((2,), (0,)), ((), ())),
                      preferred_element_type=jnp.float32)
  o_ref[...] = p.reshape(p.shape[0], 128)


def _proj(xv, w1a):
  r = xv.shape[0]
  grid = 1
  blk = r // grid
  return pl.pallas_call(
      _proj_body,
      grid=(grid,),
      in_specs=[
          pl.BlockSpec((blk, PACK, 128), lambda i: (i, 0, 0)),
          pl.BlockSpec(w1a.shape, lambda i: (0, 0)),
      ],
      out_specs=pl.BlockSpec((blk, 128), lambda i: (i, 0)),
      out_shape=jax.ShapeDtypeStruct((r, 128), jnp.float32),
  )(xv, w1a)


def _bdiag(w):
  row = jnp.concatenate([w] * PACK, axis=1)
  full = jnp.concatenate([row] * PACK, axis=0)
  si = lax.broadcasted_iota(jnp.int32, (128, 128), 0) // F_PAD
  li = lax.broadcasted_iota(jnp.int32, (128, 128), 1) // F_PAD
  return jnp.where(si == li, full, 0.0)


def _btile(b):
  return jnp.concatenate([b] * PACK, axis=1)


def _mid_body(parts_ref, b1a_ref, w1b_ref, b1b_ref, w2a_ref, o_ref):
  z = parts_ref[0] + parts_ref[1] + _btile(b1a_ref[...])
  z = jnp.maximum(z, 0.0)
  w1b = _bdiag(w1b_ref[...])
  h = jnp.dot(z, w1b, preferred_element_type=jnp.float32)
  h = jnp.maximum(h + _btile(b1b_ref[...]), 0.0)
  w2a = _bdiag(jnp.concatenate(
      [w2a_ref[...], jnp.zeros((F_PAD, 8), jnp.float32)], axis=1))
  o_ref[...] = jnp.dot(h, w2a, preferred_element_type=jnp.float32)


def _mid(partsv, b1a2, w1b, b1b2, w2a):
  r = partsv.shape[1]
  rep = lambda i: (0, 0)
  return pl.pallas_call(
      _mid_body,
      grid=(1,),
      in_specs=[
          pl.BlockSpec((2, r, 128), lambda i: (0, 0, 0)),
          pl.BlockSpec((1, F_PAD), rep),
          pl.BlockSpec((F_PAD, F_PAD), rep),
          pl.BlockSpec((1, F_PAD), rep),
          pl.BlockSpec((F_PAD, 8), rep),
      ],
      out_specs=pl.BlockSpec((r, 128), lambda i: (i, 0)),
      out_shape=jax.ShapeDtypeStruct((r, 128), jnp.float32),
  )(partsv, b1a2, w1b, b1b2, w2a)


def _tail_body(parts_ref, batch_ref, b2a_ref, w2b_ref, b2b_ref,
               wfc_ref, bfc_ref, o_ref):
  zeros18 = jnp.zeros((1, 8), jnp.float32)
  b2a16 = jnp.concatenate([b2a_ref[...], zeros18], axis=1)
  b2b16 = jnp.concatenate([b2b_ref[...], zeros18], axis=1)
  w2b16 = jnp.concatenate([
      jnp.concatenate([w2b_ref[...], jnp.zeros((8, 8), jnp.float32)], axis=1),
      jnp.zeros((8, F_PAD), jnp.float32)], axis=0)

  z = parts_ref[0] + parts_ref[1] + _btile(b2a16)
  z = jnp.maximum(z, 0.0)
  h = jnp.dot(z, _bdiag(w2b16), preferred_element_type=jnp.float32)
  h = jnp.maximum(h + _btile(b2b16), 0.0)

  r = h.shape[0]
  giota = lax.broadcasted_iota(jnp.int32, (1, 64), 1)
  ones = jnp.ones((r, 1), jnp.float32)
  sums = jnp.zeros((64, F_PAD), jnp.float32)
  cnts = jnp.zeros((64, 1), jnp.float32)
  for s in range(PACK):
    oh = (batch_ref[:, s:s + 1] == giota).astype(jnp.float32)
    sums += lax.dot_general(oh, h[:, s * F_PAD:(s + 1) * F_PAD],
                            (((0,), (0,)), ((), ())),
                            preferred_element_type=jnp.float32)
    cnts += lax.dot_general(oh, ones, (((0,), (0,)), ((), ())),
                            preferred_element_type=jnp.float32)

  pooled = sums / jnp.maximum(cnts, 1.0)
  wfc16 = jnp.concatenate(
      [wfc_ref[...], jnp.zeros((8, wfc_ref.shape[1]), jnp.float32)], axis=0)
  logits = jnp.dot(pooled, wfc16,
                   preferred_element_type=jnp.float32) + bfc_ref[...]
  m = jnp.max(logits, axis=1, keepdims=True)
  lse = jnp.log(jnp.sum(jnp.exp(logits - m), axis=1, keepdims=True)) + m
  o_ref[...] = logits - lse


def _tail(partsv, batchv, b2a2, w2b, b2b2, wfc, bfc2):
  r = partsv.shape[1]
  c = wfc.shape[1]
  full = lambda i: (0, 0)
  return pl.pallas_call(
      _tail_body,
      grid=(1,),
      in_specs=[
          pl.BlockSpec((2, r, 128), lambda i: (0, 0, 0)),
          pl.BlockSpec((r, PACK), full),
          pl.BlockSpec((1, 8), full),
          pl.BlockSpec((8, 8), full),
          pl.BlockSpec((1, 8), full),
          pl.BlockSpec((8, c), full),
          pl.BlockSpec((1, c), full),
      ],
      out_specs=pl.BlockSpec((64, c), full),
      out_shape=jax.ShapeDtypeStruct((64, c), jnp.float32),
  )(partsv, batchv, b2a2, w2b, b2b2, wfc, bfc2)


def kernel(x, edge_index, batch, W1a, b1a, W1b, b1b, W2a, b2a, W2b, b2b,
           Wfc, bfc):
  n = x.shape[0]
  e = edge_index.shape[1]
  k = 125
  r = n // PACK
  edge3d = edge_index.reshape(2, e // k, k)
  batchv = batch.reshape(r, PACK)
  xv = x.reshape(r, PACK, x.shape[1])

  p1v = _proj(xv, W1a)
  parts1 = _sc_segment_sum(p1v.reshape(n, F_PAD), edge3d)
  p2v = _mid(parts1.reshape(2, r, 128), b1a.reshape(1, -1), W1b,
             b1b.reshape(1, -1), W2a)
  parts2 = _sc_segment_sum(p2v.reshape(n, F_PAD), edge3d)
  return _tail(parts2.reshape(2, r, 128), batchv, b2a.reshape(1, -1), W2b,
               b2b.reshape(1, -1), Wfc, bfc.reshape(1, -1))
